# baseline (speedup 1.0000x reference)
import functools

import jax
import jax.numpy as jnp
from jax import lax
from jax.experimental import pallas as pl
from jax.experimental.pallas import tpu as pltpu
from jax.experimental.pallas import tpu_sc as plsc

NC = 2
NS = 16
L = 16
NW = NC * NS
BATCH = 16384
EMBED = 16
B_PER_W = BATCH // NW
CHUNK = 128
N_CHUNKS = B_PER_W // CHUNK
N_GROUPS = B_PER_W // L
POS_W = 1.0 + 0.6931471805599453


def _sc_body(u_idx, p_idx, n_idx, utab, itab, out, idx_v, rows_u, rows_p,
             rows_n, part_v, sem):
    c = lax.axis_index("c")
    s = lax.axis_index("s")
    wid = s * NC + c

    base_row = wid * N_CHUNKS
    pltpu.sync_copy(u_idx.at[pl.ds(base_row, N_CHUNKS)], idx_v.at[0])
    pltpu.sync_copy(p_idx.at[pl.ds(base_row, N_CHUNKS)], idx_v.at[1])
    pltpu.sync_copy(n_idx.at[pl.ds(base_row, N_CHUNKS)], idx_v.at[2])

    copies = []
    for t in range(N_CHUNKS):
        dst = pl.ds(t * CHUNK, CHUNK)
        copies.append(pltpu.async_copy(utab.at[idx_v.at[0, t]],
                                       rows_u.at[dst], sem))
        copies.append(pltpu.async_copy(itab.at[idx_v.at[1, t]],
                                       rows_p.at[dst], sem))
        copies.append(pltpu.async_copy(itab.at[idx_v.at[2, t]],
                                       rows_n.at[dst], sem))
    for cp in copies:
        cp.wait()

    iota = lax.iota(jnp.int32, L)
    cols = [lax.rem(iota + t, L) for t in range(L)]

    def group(g, acc):
        row = g * L + iota
        accp = jnp.zeros((L,), jnp.float32)
        accn = jnp.zeros((L,), jnp.float32)
        for t in range(L):
            gu = plsc.load_gather(rows_u, [row, cols[t]])
            gp = plsc.load_gather(rows_p, [row, cols[t]])
            gn = plsc.load_gather(rows_n, [row, cols[t]])
            accp = accp + gu * gp
            accn = accn + gu * gn
        sp = 1.0 / (1.0 + jnp.exp(-accp))
        sn = 1.0 / (1.0 + jnp.exp(-accn))
        dp = sp - 1.0
        return acc + (POS_W * (dp * dp) + sn * sn)

    part = lax.fori_loop(0, N_GROUPS, group, jnp.zeros((L,), jnp.float32))
    part_v[...] = part
    pltpu.sync_copy(part_v, out.at[wid])


_sc_call = pl.kernel(
    _sc_body,
    out_type=jax.ShapeDtypeStruct((NW, L), jnp.float32),
    mesh=plsc.VectorSubcoreMesh(core_axis_name="c", subcore_axis_name="s"),
    scratch_types=[
        pltpu.VMEM((3, N_CHUNKS, CHUNK), jnp.int32),
        pltpu.VMEM((B_PER_W, EMBED), jnp.float32),
        pltpu.VMEM((B_PER_W, EMBED), jnp.float32),
        pltpu.VMEM((B_PER_W, EMBED), jnp.float32),
        pltpu.VMEM((L,), jnp.float32),
        pltpu.SemaphoreType.DMA,
    ],
    compiler_params=pltpu.CompilerParams(needs_layout_passes=False,
                                         use_tc_tiling_on_sc=False),
)


def _reduce_body(x_ref, o_ref):
    o_ref[0, 0] = jnp.sum(x_ref[...]) * (1.0 / (2.0 * BATCH))


_reduce_call = pl.pallas_call(
    _reduce_body,
    out_shape=jax.ShapeDtypeStruct((1, 1), jnp.float32),
    out_specs=pl.BlockSpec(memory_space=pltpu.SMEM),
)


def kernel(users, positive_items, negative_items, user_embedding,
           item_embedding):
    u2 = users.astype(jnp.int32).reshape(NW * N_CHUNKS, CHUNK)
    p2 = positive_items.astype(jnp.int32).reshape(NW * N_CHUNKS, CHUNK)
    n2 = negative_items.astype(jnp.int32).reshape(NW * N_CHUNKS, CHUNK)
    partials = _sc_call(u2, p2, n2, user_embedding, item_embedding)
    return _reduce_call(partials)[0, 0]
